# baseline (speedup 1.0000x reference)
import jax
import jax.numpy as jnp
from jax.experimental import pallas as pl

_BB = 8


def _embed_kernel(x_ref, posb_ref, w_ref, o_ref):
    acc = jax.lax.dot_general(
        x_ref[...], w_ref[...], (((2,), (0,)), ((), ())),
        preferred_element_type=jnp.float32)
    o_ref[...] = acc + posb_ref[...][None, :, :]


def kernel(x, pos_table, W, b):
    B, L, D = x.shape
    posb = pos_table + b[None, :]

    return pl.pallas_call(
        _embed_kernel,
        grid=(B // _BB,),
        in_specs=[
            pl.BlockSpec((_BB, L, D), lambda i: (i, 0, 0)),
            pl.BlockSpec((L, D), lambda i: (0, 0)),
            pl.BlockSpec((D, D), lambda i: (0, 0)),
        ],
        out_specs=pl.BlockSpec((_BB, L, D), lambda i: (i, 0, 0)),
        out_shape=jax.ShapeDtypeStruct((B, L, D), x.dtype),
    )(x, posb, W)


# speedup vs baseline: 1.2699x; 1.2699x over previous
import jax
import jax.numpy as jnp
from jax.experimental import pallas as pl
from jax.experimental.pallas import tpu as pltpu

_CHUNK = 3200
_DEPTH = 4


def _in_copy(x_hbm, xbuf, in_sems, chunk, slot):
    rows = x_hbm.shape[0] * x_hbm.shape[1]
    xv = x_hbm.reshape(rows, x_hbm.shape[2])
    return pltpu.make_async_copy(
        xv.at[pl.ds(chunk * _CHUNK, _CHUNK)], xbuf.at[slot],
        in_sems.at[slot])


def _out_copy(o_hbm, obuf, out_sems, chunk, slot):
    rows = o_hbm.shape[0] * o_hbm.shape[1]
    ov = o_hbm.reshape(rows, o_hbm.shape[2])
    return pltpu.make_async_copy(
        obuf.at[slot], ov.at[pl.ds(chunk * _CHUNK, _CHUNK)],
        out_sems.at[slot])


def _embed_kernel(x_hbm, posb_ref, w_ref, o_hbm, xbuf, obuf, in_sems,
                  out_sems):
    i = pl.program_id(0)
    n = pl.num_programs(0)
    slot = jax.lax.rem(i, _DEPTH)

    @pl.when(i == 0)
    def _():
        for d in range(_DEPTH - 1):
            _in_copy(x_hbm, xbuf, in_sems, d, d).start()

    @pl.when(i + _DEPTH - 1 < n)
    def _():
        c = i + _DEPTH - 1
        _in_copy(x_hbm, xbuf, in_sems, c, jax.lax.rem(c, _DEPTH)).start()

    _in_copy(x_hbm, xbuf, in_sems, i, slot).wait()

    @pl.when(i >= _DEPTH)
    def _():
        _out_copy(o_hbm, obuf, out_sems, i - _DEPTH, slot).wait()

    acc = jax.lax.dot_general(
        xbuf[slot], w_ref[...], (((1,), (0,)), ((), ())),
        preferred_element_type=jnp.float32)
    obuf[slot] = acc + posb_ref[...]

    _out_copy(o_hbm, obuf, out_sems, i, slot).start()

    @pl.when(i == n - 1)
    def _():
        for d in range(_DEPTH):
            c = n - 1 - d
            if c >= 0:
                _out_copy(o_hbm, obuf, out_sems, c,
                          c % _DEPTH).wait()


def kernel(x, pos_table, W, b):
    B, L, D = x.shape
    rows = B * L

    posb = jnp.tile(pos_table + b[None, :], (_CHUNK // L, 1))

    out = pl.pallas_call(
        _embed_kernel,
        grid=(rows // _CHUNK,),
        in_specs=[
            pl.BlockSpec(memory_space=pltpu.MemorySpace.HBM),
            pl.BlockSpec((_CHUNK, D), lambda i: (0, 0)),
            pl.BlockSpec((D, D), lambda i: (0, 0)),
        ],
        out_specs=pl.BlockSpec(memory_space=pltpu.MemorySpace.HBM),
        out_shape=jax.ShapeDtypeStruct((B, L, D), x.dtype),
        scratch_shapes=[
            pltpu.VMEM((_DEPTH, _CHUNK, D), jnp.float32),
            pltpu.VMEM((_DEPTH, _CHUNK, D), jnp.float32),
            pltpu.SemaphoreType.DMA((_DEPTH,)),
            pltpu.SemaphoreType.DMA((_DEPTH,)),
        ],
    )(x, posb, W)
    return out


# speedup vs baseline: 3.7833x; 2.9793x over previous
import jax
import jax.numpy as jnp
from jax.experimental import pallas as pl
from jax.experimental.pallas import tpu as pltpu

_BB = 128
_PACK = 4
_DP = 128


def _embed_kernel(x_ref, posb_ref, w_ref, o_ref):
    nt = x_ref.shape[1] // _DP
    for t in range(nt):
        sl = slice(t * _DP, (t + 1) * _DP)
        acc = jax.lax.dot_general(
            x_ref[:, sl], w_ref[...], (((1,), (0,)), ((), ())),
            preferred_element_type=jnp.float32)
        o_ref[:, sl] = acc + posb_ref[:, sl]


def kernel(x, pos_table, W, b):
    B, L, D = x.shape
    F = L * D
    x2 = x.reshape(B, F)

    posb = (pos_table + b[None, :]).reshape(1, F)

    wd = jnp.zeros((_DP, _DP), dtype=W.dtype)
    for i in range(_PACK):
        wd = wd.at[i * D:(i + 1) * D, i * D:(i + 1) * D].set(W)

    out = pl.pallas_call(
        _embed_kernel,
        grid=(B // _BB,),
        in_specs=[
            pl.BlockSpec((_BB, F), lambda i: (i, 0)),
            pl.BlockSpec((1, F), lambda i: (0, 0)),
            pl.BlockSpec((_DP, _DP), lambda i: (0, 0)),
        ],
        out_specs=pl.BlockSpec((_BB, F), lambda i: (i, 0)),
        out_shape=jax.ShapeDtypeStruct((B, F), x.dtype),
    )(x2, posb, wd)
    return out.reshape(B, L, D)


# speedup vs baseline: 14.0130x; 3.7039x over previous
import jax
import jax.numpy as jnp
from jax.experimental import pallas as pl

_MB = 8


def _embed_kernel(x_ref, posb_ref, wt_ref, o_ref):
    wt = wt_ref[...]
    for t in range(x_ref.shape[0]):
        acc = jax.lax.dot_general(
            wt, x_ref[t], (((1,), (0,)), ((), ())),
            preferred_element_type=jnp.float32)
        o_ref[t] = acc + posb_ref[t][:, None]


def kernel(x, pos_table, W, b):
    B, L, D = x.shape
    xt = jnp.transpose(x, (1, 2, 0))
    posb = pos_table + b[None, :]
    wt = W.T

    out = pl.pallas_call(
        _embed_kernel,
        grid=(L // _MB,),
        in_specs=[
            pl.BlockSpec((_MB, D, B), lambda i: (i, 0, 0)),
            pl.BlockSpec((_MB, D), lambda i: (i, 0)),
            pl.BlockSpec((D, D), lambda i: (0, 0)),
        ],
        out_specs=pl.BlockSpec((_MB, D, B), lambda i: (i, 0, 0)),
        out_shape=jax.ShapeDtypeStruct((L, D, B), x.dtype),
    )(xt, posb, wt)
    return jnp.transpose(out, (2, 0, 1))


# speedup vs baseline: 14.4557x; 1.0316x over previous
import jax
import jax.numpy as jnp
from jax.experimental import pallas as pl

_MB = 20


def _embed_kernel(x_ref, posb_ref, wt_ref, o_ref):
    wt = wt_ref[...]
    base = pl.program_id(0) * _MB
    for t in range(x_ref.shape[0]):
        acc = jax.lax.dot_general(
            wt, x_ref[t], (((1,), (0,)), ((), ())),
            preferred_element_type=jnp.float32)
        o_ref[t] = acc + posb_ref[base + t][:, None]


def kernel(x, pos_table, W, b):
    B, L, D = x.shape
    xt = jnp.transpose(x, (1, 2, 0))
    posb = pos_table + b[None, :]
    wt = W.T

    out = pl.pallas_call(
        _embed_kernel,
        grid=(L // _MB,),
        in_specs=[
            pl.BlockSpec((_MB, D, B), lambda i: (i, 0, 0)),
            pl.BlockSpec((L, D), lambda i: (0, 0)),
            pl.BlockSpec((D, D), lambda i: (0, 0)),
        ],
        out_specs=pl.BlockSpec((_MB, D, B), lambda i: (i, 0, 0)),
        out_shape=jax.ShapeDtypeStruct((L, D, B), x.dtype),
    )(xt, posb, wt)
    return jnp.transpose(out, (2, 0, 1))
